# baseline (speedup 1.0000x reference)
import functools

import jax
import jax.numpy as jnp
from jax import lax
from jax.experimental import pallas as pl
from jax.experimental.pallas import tpu as pltpu
from jax.experimental.pallas import tpu_sc as plsc

_GAMMA = 12.0
_B = 16384
_D = 32
_NC = 2
_NS = 16
_NW = _NC * _NS
_BPW = _B // _NW
_L = 16
_SPAD = 17
_GRP = _BPW // _L
_NBUF = 8
_NRND = _BPW // 16

_mesh = plsc.VectorSubcoreMesh(
    core_axis_name="c", subcore_axis_name="s",
    num_cores=_NC, num_subcores=_NS)


@functools.partial(
    pl.kernel,
    out_type=jax.ShapeDtypeStruct((_NW, _BPW), jnp.float32),
    mesh=_mesh,
    compiler_params=pltpu.CompilerParams(
        needs_layout_passes=False, use_tc_tiling_on_sc=True),
    scratch_types=[
        pltpu.VMEM((_BPW,), jnp.int32),
        pltpu.VMEM((_BPW,), jnp.int32),
        pltpu.VMEM((_BPW,), jnp.int32),
        pltpu.VMEM((_BPW,), jnp.int32),
        pltpu.VMEM((_BPW,), jnp.int32),
        pltpu.VMEM((_BPW,), jnp.int32),
        pltpu.VMEM((_NBUF, _D, 128), jnp.float32),
        pltpu.VMEM((250, 128), jnp.float32),
        pltpu.VMEM((_BPW * _D,), jnp.float32),
        pltpu.VMEM((_BPW * _SPAD + _L,), jnp.float32),
        pltpu.VMEM((_BPW,), jnp.float32),
    ] + [pltpu.SemaphoreType.DMA] * _NBUF,
)
def _kg_logits(ent_t, rel_p, hcb, hcol, tcb, tcol, rq, rm, out,
               hcbv, hcolv, tcbv, tcolv, rqv, rmv,
               ring, relv, hrows, sv, ov, *sems):
    wid = lax.axis_index("s") * _NC + lax.axis_index("c")

    pltpu.sync_copy(hcb.at[wid], hcbv)
    pltpu.sync_copy(hcol.at[wid], hcolv)
    pltpu.sync_copy(tcb.at[wid], tcbv)
    pltpu.sync_copy(tcol.at[wid], tcolv)
    pltpu.sync_copy(rq.at[wid], rqv)
    pltpu.sync_copy(rm.at[wid], rmv)
    pltpu.sync_copy(rel_p, relv)

    iota = lax.iota(jnp.int32, _L)

    def fire(cb, slot):
        off = pl.multiple_of(cb * 128, 128)
        pltpu.async_copy(
            ent_t.at[:, pl.ds(off, 128)], ring.at[slot], sems[slot])

    def drain(slot):
        pltpu.make_async_copy(
            ent_t.at[:, pl.ds(0, 128)], ring.at[slot], sems[slot]).wait()

    def run_pass(cbv, vec_refs, extract):
        def load_vecs(pos):
            return tuple(v[pl.ds(pos, _L)] for v in (cbv,) + vec_refs)

        v0 = load_vecs(0)
        for l in range(_NBUF):
            fire(v0[0][l], l)
        for l in range(_NBUF):
            drain(l)
            extract(l, l, *[v[l] for v in v0[1:]])
            fire(v0[0][l + 8], l)

        def body(rnd, carry):
            cur = load_vecs(rnd * 16)
            for l in range(16):
                slot = l % _NBUF
                drain(slot)
                src, lane = (carry, l + 8) if l < 8 else (cur, l - 8)
                extract(rnd * 16 + l - _NBUF, slot, *[v[lane] for v in src[1:]])
                fire(cur[0][l], slot)
            return cur

        last = lax.fori_loop(1, _NRND, body, v0)
        base = (_NRND - 1) * 16 + _NBUF
        for l in range(_NBUF):
            drain(l)
            extract(base + l, l, *[v[l + 8] for v in last[1:]])

    def extract_h(j, slot, col):
        cspl = jnp.broadcast_to(col, (_L,))
        h0 = plsc.load_gather(ring.at[slot], [iota, cspl])
        h1 = plsc.load_gather(ring.at[slot], [iota + _L, cspl])
        plsc.store_scatter(hrows, [j * _D + iota], h0)
        plsc.store_scatter(hrows, [j * _D + _L + iota], h1)

    def extract_t(j, slot, col, q, m):
        cspl = jnp.broadcast_to(col, (_L,))
        qspl = jnp.broadcast_to(q, (_L,))
        t0 = plsc.load_gather(ring.at[slot], [iota, cspl])
        t1 = plsc.load_gather(ring.at[slot], [iota + _L, cspl])
        h0 = plsc.load_gather(hrows, [j * _D + iota])
        h1 = plsc.load_gather(hrows, [j * _D + _L + iota])
        r0 = plsc.load_gather(relv, [qspl, m + iota])
        r1 = plsc.load_gather(relv, [qspl, m + _L + iota])
        s = jnp.abs(h0 + r0 - t0) + jnp.abs(h1 + r1 - t1)
        plsc.store_scatter(sv, [j * _SPAD + iota], s)

    run_pass(hcbv, (hcolv,), extract_h)
    run_pass(tcbv, (tcolv, rqv, rmv), extract_t)

    def grp_body(g, carry):
        base = g * (_L * _SPAD)
        acc = jnp.zeros((_L,), jnp.float32)
        for j in range(_L):
            acc = acc + plsc.load_gather(sv, [base + j + iota * _SPAD])
        ov[pl.ds(g * _L, _L)] = _GAMMA - acc
        return carry

    lax.fori_loop(0, _GRP, grp_body, 0)
    pltpu.sync_copy(ov, out.at[wid])


def kernel(entity_embedding, relation_embedding, heads, relations, tails):
    ent_t = entity_embedding.T
    rel_p = relation_embedding.reshape(relation_embedding.shape[0] // 4, 128)
    h = heads.astype(jnp.int32)
    r = relations.astype(jnp.int32)
    t = tails.astype(jnp.int32)
    shape = (_NW, _BPW)
    hcb = (h >> 7).reshape(shape)
    hcol = (h & 127).reshape(shape)
    tcb = (t >> 7).reshape(shape)
    tcol = (t & 127).reshape(shape)
    rq = (r >> 2).reshape(shape)
    rm = ((r & 3) * _D).reshape(shape)
    out = _kg_logits(ent_t, rel_p, hcb, hcol, tcb, tcol, rq, rm)
    return out.reshape(_B)
